# Initial kernel scaffold; baseline (speedup 1.0000x reference)
import jax
import jax.numpy as jnp
from jax.experimental import pallas as pl


def kernel(x, edge_index, edge_weight, W1, b1, W2, b2):
    raise NotImplementedError("write your pallas kernel here")


# speedup vs baseline: 7.9580x; 7.9580x over previous
import jax
import jax.numpy as jnp
from jax import lax
from jax.experimental import pallas as pl
from jax.experimental.pallas import tpu as pltpu
from jax.experimental.pallas import tpu_sc as plsc

N = 10000
D = 128
NC = 2
NS = 16
NW = NC * NS
CHUNK = 128
L = 16

_mesh = plsc.VectorSubcoreMesh(core_axis_name="c", subcore_axis_name="s")

SLAB = 632
SLAB_LAST = N - (NS - 1) * SLAB
DEG_Z = 640


def _deg_call(colf, ewf):
    ept = colf.shape[0] // NW
    cpw = ept // CHUNK

    def body(col_hbm, ew_hbm, degp_hbm, colv, ewv, zv, deg_sh):
        cid = lax.axis_index("c")
        sid = lax.axis_index("s")
        wid = sid * NC + cid

        @pl.loop(0, DEG_Z, step=L)
        def _(i):
            zv[pl.ds(i, L)] = jnp.zeros((L,), jnp.float32)

        @pl.when(sid < NS - 1)
        def _():
            pltpu.sync_copy(zv, deg_sh.at[pl.ds(sid * DEG_Z, DEG_Z)])

        @pl.when(sid == NS - 1)
        def _():
            rem = N - (NS - 1) * DEG_Z
            pltpu.sync_copy(zv.at[pl.ds(0, rem)],
                            deg_sh.at[pl.ds((NS - 1) * DEG_Z, rem)])

        plsc.subcore_barrier()
        pltpu.sync_copy(col_hbm.at[pl.ds(wid * ept, ept)], colv)
        pltpu.sync_copy(ew_hbm.at[pl.ds(wid * ept, ept)], ewv)

        @pl.loop(0, cpw)
        def _(j):
            pltpu.sync_copy(ewv.at[pl.ds(j * CHUNK, CHUNK)],
                            deg_sh.at[colv.at[pl.ds(j * CHUNK, CHUNK)]],
                            add=True)

        plsc.subcore_barrier()

        @pl.when(sid == 0)
        def _():
            pltpu.sync_copy(deg_sh, degp_hbm.at[cid])

    k = pl.kernel(
        body,
        out_type=jax.ShapeDtypeStruct((NC, N), jnp.float32),
        mesh=_mesh,
        scratch_types=[
            pltpu.VMEM((ept,), jnp.int32),
            pltpu.VMEM((ept,), jnp.float32),
            pltpu.VMEM((DEG_Z,), jnp.float32),
            pltpu.VMEM_SHARED((N,), jnp.float32),
        ],
    )
    return k(colf, ewf)


def _agg_call(rowf, colf, ewf, xws):
    ept = rowf.shape[0] // NW
    cpw = ept // CHUNK

    def body(rows_hbm, cols_hbm, ew_hbm, xw_hbm, outp_hbm,
             rbuf, cbuf, wbuf, g0, g1, acc_sh,
             ssem0, ssem1, isem0, isem1, isem2, isem3):
        cid = lax.axis_index("c")
        sid = lax.axis_index("s")
        wid = sid * NC + cid
        ebase = wid * ept

        gbufs = (g0, g1)
        ssems = (ssem0, ssem1)
        isems = (isem0, isem1, isem2, isem3)

        @pl.loop(0, CHUNK)
        def _(i):
            for k in range(D // L):
                g0[i, pl.ds(k * L, L)] = jnp.zeros((L,), jnp.float32)

        base = sid * SLAB
        for k in range(4):
            pltpu.sync_copy(g0, acc_sh.at[pl.ds(base + k * CHUNK, CHUNK)])

        @pl.when(sid < NS - 1)
        def _():
            pltpu.sync_copy(g0.at[pl.ds(0, SLAB - 512)],
                            acc_sh.at[pl.ds(base + 512, SLAB - 512)])

        @pl.when(sid == NS - 1)
        def _():
            pltpu.sync_copy(g0.at[pl.ds(0, SLAB_LAST - 512)],
                            acc_sh.at[pl.ds(base + 512, SLAB_LAST - 512)])

        for b in range(2):
            off = ebase + b * CHUNK
            pltpu.async_copy(rows_hbm.at[pl.ds(off, CHUNK)], rbuf.at[b],
                             isems[b])
            pltpu.async_copy(cols_hbm.at[pl.ds(off, CHUNK)], cbuf.at[b],
                             isems[b])
            pltpu.async_copy(ew_hbm.at[pl.ds(off, CHUNK)], wbuf.at[b],
                             isems[b])

        plsc.subcore_barrier()

        @pl.loop(0, cpw, step=4)
        def _(j0):
            for b in range(4):
                j = j0 + b
                gb = gbufs[b % 2]
                ss = ssems[b % 2]

                if b >= 2:
                    pltpu.make_async_copy(
                        xw_hbm.at[pl.ds(0, CHUNK)], gb, ss).wait()
                else:
                    @pl.when(j0 > 0)
                    def _():
                        pltpu.make_async_copy(
                            xw_hbm.at[pl.ds(0, CHUNK)], gb, ss).wait()

                @pl.when(j + 2 < cpw)
                def _():
                    b2 = (b + 2) % 4
                    off = ebase + (j + 2) * CHUNK
                    pltpu.async_copy(rows_hbm.at[pl.ds(off, CHUNK)],
                                     rbuf.at[b2], isems[b2])
                    pltpu.async_copy(cols_hbm.at[pl.ds(off, CHUNK)],
                                     cbuf.at[b2], isems[b2])
                    pltpu.async_copy(ew_hbm.at[pl.ds(off, CHUNK)],
                                     wbuf.at[b2], isems[b2])

                pltpu.make_async_copy(
                    rows_hbm.at[pl.ds(0, CHUNK)], rbuf.at[b], isems[b]).wait()
                pltpu.make_async_copy(
                    rows_hbm.at[pl.ds(0, CHUNK)], cbuf.at[b], isems[b]).wait()
                pltpu.make_async_copy(
                    ew_hbm.at[pl.ds(0, CHUNK)], wbuf.at[b], isems[b]).wait()

                pltpu.sync_copy(xw_hbm.at[rbuf.at[b]], gb)

                @pl.loop(0, CHUNK, step=L)
                def _(e0):
                    wv = wbuf[b, pl.ds(e0, L)]
                    for i in range(L):
                        sv = jnp.broadcast_to(wv[i], (L,))
                        for k in range(D // L):
                            gb[e0 + i, pl.ds(k * L, L)] = (
                                gb[e0 + i, pl.ds(k * L, L)] * sv)

                pltpu.async_copy(gb, acc_sh.at[cbuf.at[b]], ss, add=True)

        pltpu.make_async_copy(xw_hbm.at[pl.ds(0, CHUNK)], g0, ssem0).wait()
        pltpu.make_async_copy(xw_hbm.at[pl.ds(0, CHUNK)], g1, ssem1).wait()
        plsc.subcore_barrier()

        base = sid * SLAB

        @pl.when(sid < NS - 1)
        def _():
            pltpu.sync_copy(acc_sh.at[pl.ds(base, SLAB)],
                            outp_hbm.at[cid, pl.ds(base, SLAB)])

        @pl.when(sid == NS - 1)
        def _():
            pltpu.sync_copy(acc_sh.at[pl.ds(base, SLAB_LAST)],
                            outp_hbm.at[cid, pl.ds(base, SLAB_LAST)])

    k = pl.kernel(
        body,
        out_type=jax.ShapeDtypeStruct((NC, N, D), jnp.float32),
        mesh=_mesh,
        scratch_types=[
            pltpu.VMEM((4, CHUNK), jnp.int32),
            pltpu.VMEM((4, CHUNK), jnp.int32),
            pltpu.VMEM((4, CHUNK), jnp.float32),
            pltpu.VMEM((CHUNK, D), jnp.float32),
            pltpu.VMEM((CHUNK, D), jnp.float32),
            pltpu.VMEM_SHARED((N, D), jnp.float32),
            pltpu.SemaphoreType.DMA,
            pltpu.SemaphoreType.DMA,
            pltpu.SemaphoreType.DMA,
            pltpu.SemaphoreType.DMA,
            pltpu.SemaphoreType.DMA,
            pltpu.SemaphoreType.DMA,
        ],
    )
    return k(rowf, colf, ewf, xws)


BM = 1000


def _mm_body(x_ref, w_ref, o_ref):
    o_ref[...] = jnp.dot(x_ref[...], w_ref[...],
                         preferred_element_type=jnp.float32,
                         precision=lax.Precision.HIGHEST)


def _mm_call(x, w):
    return pl.pallas_call(
        _mm_body,
        grid=(N // BM,),
        in_specs=[pl.BlockSpec((BM, D), lambda i: (i, 0)),
                  pl.BlockSpec((D, D), lambda i: (0, 0))],
        out_specs=pl.BlockSpec((BM, D), lambda i: (i, 0)),
        out_shape=jax.ShapeDtypeStruct((N, D), jnp.float32),
    )(x, w)


def _scale_body(xw_ref, degt_ref, dis_ref, xws_ref):
    deg = degt_ref[:, 0:1] + degt_ref[:, 1:2] + 1.0
    dis = lax.rsqrt(deg)
    dis_ref[...] = dis
    xws_ref[...] = xw_ref[...] * dis


def _scale_call(xw, degt):
    return pl.pallas_call(
        _scale_body,
        grid=(N // BM,),
        in_specs=[pl.BlockSpec((BM, D), lambda i: (i, 0)),
                  pl.BlockSpec((BM, NC), lambda i: (i, 0))],
        out_specs=[pl.BlockSpec((BM, 1), lambda i: (i, 0)),
                   pl.BlockSpec((BM, D), lambda i: (i, 0))],
        out_shape=[jax.ShapeDtypeStruct((N, 1), jnp.float32),
                   jax.ShapeDtypeStruct((N, D), jnp.float32)],
    )(xw, degt)


def _mid_body(op_ref, xws_ref, dis_ref, b_ref, w_ref, o_ref):
    h = dis_ref[...] * (op_ref[0] + op_ref[1] + xws_ref[...]) + b_ref[...]
    h = jnp.maximum(h, 0.0)
    o_ref[...] = jnp.dot(h, w_ref[...],
                         preferred_element_type=jnp.float32,
                         precision=lax.Precision.HIGHEST) * dis_ref[...]


def _mid_call(outp, xws, dis, b, w):
    return pl.pallas_call(
        _mid_body,
        grid=(N // BM,),
        in_specs=[pl.BlockSpec((NC, BM, D), lambda i: (0, i, 0)),
                  pl.BlockSpec((BM, D), lambda i: (i, 0)),
                  pl.BlockSpec((BM, 1), lambda i: (i, 0)),
                  pl.BlockSpec((1, D), lambda i: (0, 0)),
                  pl.BlockSpec((D, D), lambda i: (0, 0))],
        out_specs=pl.BlockSpec((BM, D), lambda i: (i, 0)),
        out_shape=jax.ShapeDtypeStruct((N, D), jnp.float32),
    )(outp, xws, dis, b, w)


def _fin_body(op_ref, xws_ref, dis_ref, b_ref, o_ref):
    o_ref[...] = (dis_ref[...] * (op_ref[0] + op_ref[1] + xws_ref[...])
                  + b_ref[...])


def _fin_call(outp, xws, dis, b):
    return pl.pallas_call(
        _fin_body,
        grid=(N // BM,),
        in_specs=[pl.BlockSpec((NC, BM, D), lambda i: (0, i, 0)),
                  pl.BlockSpec((BM, D), lambda i: (i, 0)),
                  pl.BlockSpec((BM, 1), lambda i: (i, 0)),
                  pl.BlockSpec((1, D), lambda i: (0, 0))],
        out_specs=pl.BlockSpec((BM, D), lambda i: (i, 0)),
        out_shape=jax.ShapeDtypeStruct((N, D), jnp.float32),
    )(outp, xws, dis, b)


def kernel(x, edge_index, edge_weight, W1, b1, W2, b2):
    row = edge_index[0].astype(jnp.int32)
    col = edge_index[1].astype(jnp.int32)
    ew = edge_weight.astype(jnp.float32)
    e = row.shape[0]

    epw = -(-e // NW)
    cpw = -(-epw // CHUNK)
    cpw = ((cpw + 3) // 4) * 4
    etot = NW * cpw * CHUNK
    rowf = jnp.pad(row, (0, etot - e))
    colf = jnp.pad(col, (0, etot - e))
    ewf = jnp.pad(ew, (0, etot - e))

    degp = _deg_call(colf, ewf)
    xw1 = _mm_call(x, W1)
    dis2, xw1s = _scale_call(xw1, degp.T)
    outp1 = _agg_call(rowf, colf, ewf, xw1s)
    xw2s = _mid_call(outp1, xw1s, dis2, b1.reshape(1, D), W2)
    outp2 = _agg_call(rowf, colf, ewf, xw2s)
    return _fin_call(outp2, xw2s, dis2, b2.reshape(1, D))
